# baseline (speedup 1.0000x reference)
import jax
import jax.numpy as jnp
from jax.experimental import pallas as pl
from jax.experimental.pallas import tpu as pltpu

_EPS = 1e-05


def _stats_body(x_ref, sum_ref, sq_ref):
    s = pl.program_id(1)
    xb = x_ref[...]
    bsum = jnp.sum(xb, axis=0, keepdims=True)
    bsq = jnp.sum(xb * xb, axis=0, keepdims=True)

    @pl.when(s == 0)
    def _():
        sum_ref[0] = bsum
        sq_ref[0] = bsq

    @pl.when(s != 0)
    def _():
        sum_ref[0] += bsum
        sq_ref[0] += bsq


def _make_norm_body(n_total):
    n = float(n_total)

    def _norm_body(x_ref, sum_ref, sq_ref, count_ref, mean_ref, m2_ref, o_ref):
        tot = jnp.sum(sum_ref[...], axis=0)
        totsq = jnp.sum(sq_ref[...], axis=0)
        count = count_ref[...]
        mean = mean_ref[...]
        m2 = m2_ref[...]

        new_count = count + n
        mean_b = tot / n
        m2_b = totsq - mean_b * mean_b * n
        delta = mean_b - mean
        new_mean = mean + delta * (n / new_count)
        new_m2 = m2 + m2_b + delta * delta * (count * n / new_count)
        denom = jnp.maximum(new_count - 1.0, 1.0)
        inv_std = jax.lax.rsqrt(new_m2 / denom + _EPS)
        o_ref[...] = (x_ref[...] - new_mean) * inv_std

    return _norm_body


def _row_split(rows, max_block_rows):
    for s in range(1, rows + 1):
        if rows % s == 0:
            br = rows // s
            if br % 8 == 0 and br <= max_block_rows:
                return br, s
    return rows, 1


def kernel(x, count, mean, m2):
    n_total, d = x.shape
    p1 = 2 if n_total % 2 == 0 else 1
    rows1 = n_total // p1
    br1, s1 = _row_split(rows1, 50_000)
    partial_shape = jax.ShapeDtypeStruct((p1, 1, d), jnp.float32)
    psum, psq = pl.pallas_call(
        _stats_body,
        grid=(p1, s1),
        in_specs=[pl.BlockSpec((br1, d), lambda i, s: (i * s1 + s, 0))],
        out_specs=[
            pl.BlockSpec((1, 1, d), lambda i, s: (i, 0, 0)),
            pl.BlockSpec((1, 1, d), lambda i, s: (i, 0, 0)),
        ],
        out_shape=[partial_shape, partial_shape],
        compiler_params=pltpu.CompilerParams(
            dimension_semantics=("parallel", "arbitrary"),
            vmem_limit_bytes=50 * 1024 * 1024,
        ),
        name="vnorm_stats",
    )(x)

    br2 = 30_400
    s2 = -(-n_total // br2)
    count2 = count.reshape(1, d)
    mean2 = mean.reshape(1, d)
    m22 = m2.reshape(1, d)
    vec_spec = pl.BlockSpec((1, d), lambda i: (0, 0))
    part_spec = pl.BlockSpec((p1, 1, d), lambda i: (0, 0, 0))
    out = pl.pallas_call(
        _make_norm_body(n_total),
        grid=(s2,),
        in_specs=[
            pl.BlockSpec((br2, d), lambda i: (i, 0)),
            part_spec,
            part_spec,
            vec_spec,
            vec_spec,
            vec_spec,
        ],
        out_specs=pl.BlockSpec((br2, d), lambda i: (i, 0)),
        out_shape=jax.ShapeDtypeStruct((n_total, d), jnp.float32),
        compiler_params=pltpu.CompilerParams(
            dimension_semantics=("parallel",),
            vmem_limit_bytes=61 * 1024 * 1024,
        ),
        name="vnorm_apply",
    )(x, psum, psq, count2, mean2, m22)
    return out


# speedup vs baseline: 1.0009x; 1.0009x over previous
import jax
import jax.numpy as jnp
from jax.experimental import pallas as pl
from jax.experimental.pallas import tpu as pltpu

_EPS = 1e-05


def _stats_body(x_ref, sum_ref, sq_ref):
    s = pl.program_id(1)
    xb = x_ref[...]
    bsum = jnp.sum(xb, axis=0, keepdims=True)
    bsq = jnp.sum(xb * xb, axis=0, keepdims=True)

    @pl.when(s == 0)
    def _():
        sum_ref[0] = bsum
        sq_ref[0] = bsq

    @pl.when(s != 0)
    def _():
        sum_ref[0] += bsum
        sq_ref[0] += bsq


def _make_norm_body(n_total):
    n = float(n_total)

    def _norm_body(x_ref, sum_ref, sq_ref, count_ref, mean_ref, m2_ref, o_ref):
        tot = jnp.sum(sum_ref[...], axis=0)
        totsq = jnp.sum(sq_ref[...], axis=0)
        count = count_ref[...]
        mean = mean_ref[...]
        m2 = m2_ref[...]

        new_count = count + n
        mean_b = tot / n
        m2_b = totsq - mean_b * mean_b * n
        delta = mean_b - mean
        new_mean = mean + delta * (n / new_count)
        new_m2 = m2 + m2_b + delta * delta * (count * n / new_count)
        denom = jnp.maximum(new_count - 1.0, 1.0)
        inv_std = jax.lax.rsqrt(new_m2 / denom + _EPS)
        o_ref[...] = (x_ref[...] - new_mean) * inv_std

    return _norm_body


def _row_split(rows, max_block_rows):
    for s in range(1, rows + 1):
        if rows % s == 0:
            br = rows // s
            if br % 8 == 0 and br <= max_block_rows:
                return br, s
    return rows, 1


def kernel(x, count, mean, m2):
    n_total, d = x.shape
    p1 = 2 if n_total % 2 == 0 else 1
    rows1 = n_total // p1
    br1, s1 = _row_split(rows1, 50_000)
    partial_shape = jax.ShapeDtypeStruct((p1, 1, d), jnp.float32)
    psum, psq = pl.pallas_call(
        _stats_body,
        grid=(p1, s1),
        in_specs=[pl.BlockSpec((br1, d), lambda i, s: (i * s1 + s, 0))],
        out_specs=[
            pl.BlockSpec((1, 1, d), lambda i, s: (i, 0, 0)),
            pl.BlockSpec((1, 1, d), lambda i, s: (i, 0, 0)),
        ],
        out_shape=[partial_shape, partial_shape],
        compiler_params=pltpu.CompilerParams(
            dimension_semantics=("parallel", "arbitrary"),
            vmem_limit_bytes=50 * 1024 * 1024,
        ),
        name="vnorm_stats",
    )(x)

    br2 = 28_800
    s2 = -(-n_total // br2)
    count2 = count.reshape(1, d)
    mean2 = mean.reshape(1, d)
    m22 = m2.reshape(1, d)
    vec_spec = pl.BlockSpec((1, d), lambda i: (0, 0))
    part_spec = pl.BlockSpec((p1, 1, d), lambda i: (0, 0, 0))
    out = pl.pallas_call(
        _make_norm_body(n_total),
        grid=(s2,),
        in_specs=[
            pl.BlockSpec((br2, d), lambda i: (i, 0)),
            part_spec,
            part_spec,
            vec_spec,
            vec_spec,
            vec_spec,
        ],
        out_specs=pl.BlockSpec((br2, d), lambda i: (i, 0)),
        out_shape=jax.ShapeDtypeStruct((n_total, d), jnp.float32),
        compiler_params=pltpu.CompilerParams(
            dimension_semantics=("parallel",),
            vmem_limit_bytes=58 * 1024 * 1024,
        ),
        name="vnorm_apply",
    )(x, psum, psq, count2, mean2, m22)
    return out
